# baseline (speedup 1.0000x reference)
import functools

import jax
import jax.numpy as jnp
from jax import lax
from jax.experimental import pallas as pl
from jax.experimental.pallas import tpu as pltpu
from jax.experimental.pallas import tpu_sc as plsc

_BATCH = 16384
_EMB = 32
_NROWS = 100000
_HALF0 = 50048
_HALF1 = _NROWS - _HALF0
_NC, _NS = 2, 16
_GRP = 16
_UNROLL = 16

_mesh = plsc.VectorSubcoreMesh(
    core_axis_name="c", subcore_axis_name="s", num_cores=_NC, num_subcores=_NS
)


@functools.partial(
    pl.kernel,
    out_type=jax.ShapeDtypeStruct((2 * _EMB, _BATCH), jnp.float32),
    mesh=_mesh,
    scratch_types=(
        pltpu.VMEM((_NROWS,), jnp.float32),
        pltpu.VMEM((_BATCH,), jnp.int32),
        pltpu.VMEM((_BATCH // 2,), jnp.float32),
        pltpu.SemaphoreType.DMA,
        pltpu.SemaphoreType.DMA,
        pltpu.SemaphoreType.DMA,
    ),
    compiler_params=pltpu.CompilerParams(needs_layout_passes=False),
)
def _sc_gather_t(ut, it, uids, iids, out_t, rowbuf, idxbuf, outbuf, s0, s1, s2):
    wid = lax.axis_index("s") * _NC + lax.axis_index("c")
    half_b = _BATCH // 2
    for half, (tab, ids) in enumerate(((ut, uids), (it, iids))):
        c0 = pltpu.async_copy(tab.at[wid], rowbuf, s0)
        ci = pltpu.async_copy(ids, idxbuf, s2)
        ci.wait()
        c0.wait()
        for c in range(2):
            base = c * half_b

            def body(g, carry, base=base):
                for u in range(_UNROLL):
                    off = (g * _UNROLL + u) * _GRP
                    idx = idxbuf[pl.ds(base + off, _GRP)]
                    outbuf[pl.ds(off, _GRP)] = plsc.load_gather(rowbuf, [idx])
                return carry

            lax.fori_loop(0, half_b // (_GRP * _UNROLL), body, 0)
            pltpu.sync_copy(outbuf, out_t.at[half * _EMB + wid, pl.ds(base, half_b)])


_GRID = 4
_BLK = _BATCH // _GRID


def _fm_body(feat_ref, lab_ref, vt_ref, pred_ref, mse_ref, obj_ref):
    g = pl.program_id(0)
    feat = feat_ref[...]
    vt = vt_ref[...]
    a = jnp.dot(vt, feat, preferred_element_type=jnp.float32)
    t1 = jnp.sum(a * a, axis=0)
    w = jnp.sum(vt * vt, axis=0)
    t2 = jnp.sum(feat * feat * w[:, None], axis=0)
    pred = 0.5 * (t1 - t2)
    mse = jnp.square(pred - lab_ref[...])
    pred_ref[...] = pred
    mse_ref[...] = mse

    @pl.when(g == 0)
    def _():
        obj_ref[0, 0] = 0.0

    obj_ref[0, 0] += jnp.sum(mse) * (1.0 / _BATCH)


_fm_call = pl.pallas_call(
    _fm_body,
    grid=(_GRID,),
    out_shape=(
        jax.ShapeDtypeStruct((_BATCH,), jnp.float32),
        jax.ShapeDtypeStruct((_BATCH,), jnp.float32),
        jax.ShapeDtypeStruct((1, 1), jnp.float32),
    ),
    in_specs=[
        pl.BlockSpec((2 * _EMB, _BLK), lambda g: (0, g)),
        pl.BlockSpec((_BLK,), lambda g: (g,)),
        pl.BlockSpec((16, 2 * _EMB), lambda g: (0, 0)),
    ],
    out_specs=(
        pl.BlockSpec((_BLK,), lambda g: (g,)),
        pl.BlockSpec((_BLK,), lambda g: (g,)),
        pl.BlockSpec(memory_space=pltpu.SMEM),
    ),
)


def kernel(uids, iids, labels, user_emb, item_emb, fc_W, fc_b, fm_V):
    del fc_W, fc_b
    feat_t = _sc_gather_t(
        user_emb.T, item_emb.T, uids.astype(jnp.int32), iids.astype(jnp.int32)
    )
    vt = jnp.zeros((16, 2 * _EMB), jnp.float32).at[:10, :].set(fm_V.T)
    pred, mse, obj = _fm_call(feat_t, labels, vt)
    return pred, obj[0, 0], mse


# speedup vs baseline: 1.0018x; 1.0018x over previous
import functools

import jax
import jax.numpy as jnp
from jax import lax
from jax.experimental import pallas as pl
from jax.experimental.pallas import tpu as pltpu
from jax.experimental.pallas import tpu_sc as plsc

_BATCH = 16384
_EMB = 32
_NROWS = 100000
_HALF0 = 50048
_HALF1 = _NROWS - _HALF0
_NC, _NS = 2, 16
_GRP = 16
_UNROLL = 16

_mesh = plsc.VectorSubcoreMesh(
    core_axis_name="c", subcore_axis_name="s", num_cores=_NC, num_subcores=_NS
)


@functools.partial(
    pl.kernel,
    out_type=jax.ShapeDtypeStruct((2 * _EMB, _BATCH), jnp.float32),
    mesh=_mesh,
    scratch_types=(
        pltpu.VMEM((_NROWS,), jnp.float32),
        pltpu.VMEM((_BATCH,), jnp.int32),
        pltpu.VMEM((_BATCH // 2,), jnp.float32),
        pltpu.SemaphoreType.DMA,
        pltpu.SemaphoreType.DMA,
        pltpu.SemaphoreType.DMA,
    ),
    compiler_params=pltpu.CompilerParams(needs_layout_passes=False),
)
def _sc_gather_t(ut, it, uids, iids, out_t, rowbuf, idxbuf, outbuf, s0, s1, s2):
    wid = lax.axis_index("s") * _NC + lax.axis_index("c")
    half_b = _BATCH // 2
    for half, (tab, ids) in enumerate(((ut, uids), (it, iids))):
        c0 = pltpu.async_copy(tab.at[wid], rowbuf, s0)
        ci = pltpu.async_copy(ids, idxbuf, s2)
        ci.wait()
        c0.wait()
        for c in range(2):
            base = c * half_b

            def body(g, carry, base=base):
                for u in range(_UNROLL):
                    off = (g * _UNROLL + u) * _GRP
                    idx = idxbuf[pl.ds(base + off, _GRP)]
                    outbuf[pl.ds(off, _GRP)] = plsc.load_gather(rowbuf, [idx])
                return carry

            lax.fori_loop(0, half_b // (_GRP * _UNROLL), body, 0)
            pltpu.sync_copy(outbuf, out_t.at[half * _EMB + wid, pl.ds(base, half_b)])


_GRID = 4
_BLK = _BATCH // _GRID


def _fm_body(feat_ref, lab_ref, vt_ref, pred_ref, mse_ref, obj_ref):
    g = pl.program_id(0)
    feat = feat_ref[...]
    vt = vt_ref[...]
    a = jnp.dot(
        vt.astype(jnp.bfloat16),
        feat.astype(jnp.bfloat16),
        preferred_element_type=jnp.float32,
    )
    t1 = jnp.sum(a * a, axis=0)
    w = jnp.sum(vt * vt, axis=0)
    t2 = jnp.sum(feat * feat * w[:, None], axis=0)
    pred = 0.5 * (t1 - t2)
    mse = jnp.square(pred - lab_ref[...])
    pred_ref[...] = pred
    mse_ref[...] = mse

    @pl.when(g == 0)
    def _():
        obj_ref[0, 0] = 0.0

    obj_ref[0, 0] += jnp.sum(mse) * (1.0 / _BATCH)


_fm_call = pl.pallas_call(
    _fm_body,
    grid=(_GRID,),
    out_shape=(
        jax.ShapeDtypeStruct((_BATCH,), jnp.float32),
        jax.ShapeDtypeStruct((_BATCH,), jnp.float32),
        jax.ShapeDtypeStruct((1, 1), jnp.float32),
    ),
    in_specs=[
        pl.BlockSpec((2 * _EMB, _BLK), lambda g: (0, g)),
        pl.BlockSpec((_BLK,), lambda g: (g,)),
        pl.BlockSpec((16, 2 * _EMB), lambda g: (0, 0)),
    ],
    out_specs=(
        pl.BlockSpec((_BLK,), lambda g: (g,)),
        pl.BlockSpec((_BLK,), lambda g: (g,)),
        pl.BlockSpec(memory_space=pltpu.SMEM),
    ),
)


def kernel(uids, iids, labels, user_emb, item_emb, fc_W, fc_b, fm_V):
    del fc_W, fc_b
    feat_t = _sc_gather_t(
        user_emb.T, item_emb.T, uids.astype(jnp.int32), iids.astype(jnp.int32)
    )
    vt = jnp.zeros((16, 2 * _EMB), jnp.float32).at[:10, :].set(fm_V.T)
    pred, mse, obj = _fm_call(feat_t, labels, vt)
    return pred, obj[0, 0], mse


# speedup vs baseline: 1.0026x; 1.0008x over previous
import functools

import jax
import jax.numpy as jnp
from jax import lax
from jax.experimental import pallas as pl
from jax.experimental.pallas import tpu as pltpu
from jax.experimental.pallas import tpu_sc as plsc

_BATCH = 16384
_EMB = 32
_NROWS = 100000
_HALF0 = 50048
_HALF1 = _NROWS - _HALF0
_NC, _NS = 2, 16
_GRP = 16
_UNROLL = 16

_mesh = plsc.VectorSubcoreMesh(
    core_axis_name="c", subcore_axis_name="s", num_cores=_NC, num_subcores=_NS
)


@functools.partial(
    pl.kernel,
    out_type=jax.ShapeDtypeStruct((2 * _EMB, _BATCH), jnp.float32),
    mesh=_mesh,
    scratch_types=(
        pltpu.VMEM((_NROWS,), jnp.float32),
        pltpu.VMEM((_BATCH,), jnp.int32),
        pltpu.VMEM((_BATCH // 2,), jnp.float32),
        pltpu.SemaphoreType.DMA,
        pltpu.SemaphoreType.DMA,
        pltpu.SemaphoreType.DMA,
    ),
    compiler_params=pltpu.CompilerParams(needs_layout_passes=False),
)
def _sc_gather_t(ut, it, uids, iids, out_t, rowbuf, idxbuf, outbuf, s0, s1, s2):
    wid = lax.axis_index("s") * _NC + lax.axis_index("c")
    half_b = _BATCH // 2
    for half, (tab, ids) in enumerate(((ut, uids), (it, iids))):
        c0 = pltpu.async_copy(tab.at[wid], rowbuf, s0)
        ci = pltpu.async_copy(ids, idxbuf, s2)
        ci.wait()
        c0.wait()
        for c in range(2):
            base = c * half_b

            def body(g, carry, base=base):
                for u in range(_UNROLL):
                    off = (g * _UNROLL + u) * _GRP
                    idx = idxbuf[pl.ds(base + off, _GRP)]
                    outbuf[pl.ds(off, _GRP)] = plsc.load_gather(rowbuf, [idx])
                return carry

            lax.fori_loop(0, half_b // (_GRP * _UNROLL), body, 0)
            pltpu.sync_copy(outbuf, out_t.at[half * _EMB + wid, pl.ds(base, half_b)])


_GRID = 4
_BLK = _BATCH // _GRID


def _fm_body(feat_ref, lab_ref, vt_ref, pred_ref, mse_ref, obj_ref):
    g = pl.program_id(0)
    feat = feat_ref[...]
    vt = vt_ref[...]
    a = jnp.dot(vt, feat, preferred_element_type=jnp.float32)
    t1 = jnp.sum(a * a, axis=0)
    w = jnp.sum(vt * vt, axis=0)
    t2 = jnp.sum(feat * feat * w[:, None], axis=0)
    pred = 0.5 * (t1 - t2)
    mse = jnp.square(pred - lab_ref[...])
    pred_ref[...] = pred
    mse_ref[...] = mse

    @pl.when(g == 0)
    def _():
        obj_ref[0, 0] = 0.0

    obj_ref[0, 0] += jnp.sum(mse) * (1.0 / _BATCH)


_fm_call = pl.pallas_call(
    _fm_body,
    grid=(_GRID,),
    out_shape=(
        jax.ShapeDtypeStruct((_BATCH,), jnp.float32),
        jax.ShapeDtypeStruct((_BATCH,), jnp.float32),
        jax.ShapeDtypeStruct((1, 1), jnp.float32),
    ),
    in_specs=[
        pl.BlockSpec((2 * _EMB, _BLK), lambda g: (0, g)),
        pl.BlockSpec((_BLK,), lambda g: (g,)),
        pl.BlockSpec((16, 2 * _EMB), lambda g: (0, 0)),
    ],
    out_specs=(
        pl.BlockSpec((_BLK,), lambda g: (g,)),
        pl.BlockSpec((_BLK,), lambda g: (g,)),
        pl.BlockSpec(memory_space=pltpu.SMEM),
    ),
)


def kernel(uids, iids, labels, user_emb, item_emb, fc_W, fc_b, fm_V):
    del fc_W, fc_b
    feat_t = _sc_gather_t(
        user_emb.T, item_emb.T, uids.astype(jnp.int32), iids.astype(jnp.int32)
    )
    vt = jnp.zeros((16, 2 * _EMB), jnp.float32).at[:10, :].set(fm_V.T)
    pred, mse, obj = _fm_call(feat_t, labels, vt)
    return pred, obj[0, 0], mse


# speedup vs baseline: 1.1662x; 1.1631x over previous
import functools

import jax
import jax.numpy as jnp
from jax import lax
from jax.experimental import pallas as pl
from jax.experimental.pallas import tpu as pltpu
from jax.experimental.pallas import tpu_sc as plsc

_BATCH = 16384
_EMB = 32
_NROWS = 100000
_HALF0 = 50048
_HALF1 = _NROWS - _HALF0
_NC, _NS = 2, 16
_GRP = 16
_UNROLL = 16

_mesh = plsc.VectorSubcoreMesh(
    core_axis_name="c", subcore_axis_name="s", num_cores=_NC, num_subcores=_NS
)


@functools.partial(
    pl.kernel,
    out_type=jax.ShapeDtypeStruct((2 * _EMB, _BATCH), jnp.float32),
    mesh=_mesh,
    scratch_types=(
        pltpu.VMEM((_NROWS,), jnp.float32),
        pltpu.VMEM((_BATCH,), jnp.int32),
        pltpu.VMEM((_BATCH // 4,), jnp.float32),
        pltpu.VMEM((_BATCH // 4,), jnp.float32),
        pltpu.SemaphoreType.DMA,
        pltpu.SemaphoreType.DMA,
        pltpu.SemaphoreType.DMA,
    ),
    compiler_params=pltpu.CompilerParams(needs_layout_passes=False),
)
def _sc_gather_t(ut, it, uids, iids, out_t, rowbuf, idxbuf, outa, outb, s0, s1, s2):
    wid = lax.axis_index("s") * _NC + lax.axis_index("c")
    chunk = _BATCH // 4
    obufs = (outa, outb)
    osems = (s1, s2)
    for half, (tab, ids) in enumerate(((ut, uids), (it, iids))):
        c0 = pltpu.async_copy(tab.at[wid], rowbuf, s0)
        ci = pltpu.async_copy(ids, idxbuf, s1)
        ci.wait()
        c0.wait()
        writes = [None, None]
        for c in range(4):
            base = c * chunk
            obuf = obufs[c % 2]
            if writes[c % 2] is not None:
                writes[c % 2].wait()

            def body(g, carry, base=base, obuf=obuf):
                idxs = []
                for u in range(_UNROLL):
                    off = (g * _UNROLL + u) * _GRP
                    idxs.append(idxbuf[pl.ds(base + off, _GRP)])
                vals = [plsc.load_gather(rowbuf, [idx]) for idx in idxs]
                for u in range(_UNROLL):
                    off = (g * _UNROLL + u) * _GRP
                    obuf[pl.ds(off, _GRP)] = vals[u]
                return carry

            lax.fori_loop(0, chunk // (_GRP * _UNROLL), body, 0)
            writes[c % 2] = pltpu.async_copy(
                obuf, out_t.at[half * _EMB + wid, pl.ds(base, chunk)], osems[c % 2]
            )
        writes[0].wait()
        writes[1].wait()


_GRID = 2
_BLK = _BATCH // _GRID


def _fm_body(feat_ref, lab_ref, vt_ref, pred_ref, mse_ref, obj_ref):
    g = pl.program_id(0)
    feat = feat_ref[...]
    vt = vt_ref[...]
    a = jnp.dot(vt, feat, preferred_element_type=jnp.float32)
    t1 = jnp.sum(a * a, axis=0)
    w = jnp.sum(vt * vt, axis=0)
    t2 = jnp.sum(feat * feat * w[:, None], axis=0)
    pred = 0.5 * (t1 - t2)
    mse = jnp.square(pred - lab_ref[...])
    pred_ref[...] = pred
    mse_ref[...] = mse

    @pl.when(g == 0)
    def _():
        obj_ref[0, 0] = 0.0

    obj_ref[0, 0] += jnp.sum(mse) * (1.0 / _BATCH)


_fm_call = pl.pallas_call(
    _fm_body,
    grid=(_GRID,),
    out_shape=(
        jax.ShapeDtypeStruct((_BATCH,), jnp.float32),
        jax.ShapeDtypeStruct((_BATCH,), jnp.float32),
        jax.ShapeDtypeStruct((1, 1), jnp.float32),
    ),
    in_specs=[
        pl.BlockSpec((2 * _EMB, _BLK), lambda g: (0, g)),
        pl.BlockSpec((_BLK,), lambda g: (g,)),
        pl.BlockSpec((16, 2 * _EMB), lambda g: (0, 0)),
    ],
    out_specs=(
        pl.BlockSpec((_BLK,), lambda g: (g,)),
        pl.BlockSpec((_BLK,), lambda g: (g,)),
        pl.BlockSpec(memory_space=pltpu.SMEM),
    ),
)


def kernel(uids, iids, labels, user_emb, item_emb, fc_W, fc_b, fm_V):
    del fc_W, fc_b
    feat_t = _sc_gather_t(
        user_emb.T, item_emb.T, uids.astype(jnp.int32), iids.astype(jnp.int32)
    )
    vt = jnp.zeros((16, 2 * _EMB), jnp.float32).at[:10, :].set(fm_V.T)
    pred, mse, obj = _fm_call(feat_t, labels, vt)
    return pred, obj[0, 0], mse


# speedup vs baseline: 1.1690x; 1.0025x over previous
import functools

import jax
import jax.numpy as jnp
from jax import lax
from jax.experimental import pallas as pl
from jax.experimental.pallas import tpu as pltpu
from jax.experimental.pallas import tpu_sc as plsc

_BATCH = 16384
_EMB = 32
_NROWS = 100000
_HALF0 = 50048
_HALF1 = _NROWS - _HALF0
_NC, _NS = 2, 16
_GRP = 16
_UNROLL = 16

_mesh = plsc.VectorSubcoreMesh(
    core_axis_name="c", subcore_axis_name="s", num_cores=_NC, num_subcores=_NS
)


@functools.partial(
    pl.kernel,
    out_type=jax.ShapeDtypeStruct((2 * _EMB, _BATCH), jnp.float32),
    mesh=_mesh,
    scratch_types=(
        pltpu.VMEM((_NROWS,), jnp.float32),
        pltpu.VMEM((_BATCH,), jnp.int32),
        pltpu.VMEM((_BATCH // 4,), jnp.float32),
        pltpu.VMEM((_BATCH // 4,), jnp.float32),
        pltpu.SemaphoreType.DMA,
        pltpu.SemaphoreType.DMA,
        pltpu.SemaphoreType.DMA,
    ),
    compiler_params=pltpu.CompilerParams(needs_layout_passes=False),
)
def _sc_gather_t(ut, it, uids, iids, out_t, rowbuf, idxbuf, outa, outb, s0, s1, s2):
    wid = lax.axis_index("s") * _NC + lax.axis_index("c")
    chunk = _BATCH // 4
    obufs = (outa, outb)
    osems = (s1, s2)
    for half, (tab, ids) in enumerate(((ut, uids), (it, iids))):
        c0 = pltpu.async_copy(tab.at[wid], rowbuf, s0)
        ci = pltpu.async_copy(ids, idxbuf, s1)
        ci.wait()
        c0.wait()
        writes = [None, None]
        for c in range(4):
            base = c * chunk
            obuf = obufs[c % 2]
            if writes[c % 2] is not None:
                writes[c % 2].wait()

            def body(g, carry, base=base, obuf=obuf):
                idxs = []
                for u in range(_UNROLL):
                    off = (g * _UNROLL + u) * _GRP
                    idxs.append(idxbuf[pl.ds(base + off, _GRP)])
                vals = [plsc.load_gather(rowbuf, [idx]) for idx in idxs]
                for u in range(_UNROLL):
                    off = (g * _UNROLL + u) * _GRP
                    obuf[pl.ds(off, _GRP)] = vals[u]
                return carry

            lax.fori_loop(0, chunk // (_GRP * _UNROLL), body, 0)
            writes[c % 2] = pltpu.async_copy(
                obuf, out_t.at[half * _EMB + wid, pl.ds(base, chunk)], osems[c % 2]
            )
        writes[0].wait()
        writes[1].wait()


_GRID = 2
_BLK = _BATCH // _GRID


def _fm_body(feat_ref, lab_ref, v_ref, pred_ref, mse_ref, obj_ref):
    g = pl.program_id(0)
    feat = feat_ref[...]
    v = v_ref[...]
    a = lax.dot_general(
        v, feat, (((0,), (0,)), ((), ())), preferred_element_type=jnp.float32
    )
    t1 = jnp.sum(a * a, axis=0)
    w = jnp.sum(v * v, axis=1)
    t2 = jnp.sum(feat * feat * w[:, None], axis=0)
    pred = 0.5 * (t1 - t2)
    mse = jnp.square(pred - lab_ref[...])
    pred_ref[...] = pred
    mse_ref[...] = mse

    @pl.when(g == 0)
    def _():
        obj_ref[0, 0] = 0.0

    obj_ref[0, 0] += jnp.sum(mse) * (1.0 / _BATCH)


_fm_call = pl.pallas_call(
    _fm_body,
    grid=(_GRID,),
    out_shape=(
        jax.ShapeDtypeStruct((_BATCH,), jnp.float32),
        jax.ShapeDtypeStruct((_BATCH,), jnp.float32),
        jax.ShapeDtypeStruct((1, 1), jnp.float32),
    ),
    in_specs=[
        pl.BlockSpec((2 * _EMB, _BLK), lambda g: (0, g)),
        pl.BlockSpec((_BLK,), lambda g: (g,)),
        pl.BlockSpec((2 * _EMB, 10), lambda g: (0, 0)),
    ],
    out_specs=(
        pl.BlockSpec((_BLK,), lambda g: (g,)),
        pl.BlockSpec((_BLK,), lambda g: (g,)),
        pl.BlockSpec(memory_space=pltpu.SMEM),
    ),
)


def kernel(uids, iids, labels, user_emb, item_emb, fc_W, fc_b, fm_V):
    del fc_W, fc_b
    feat_t = _sc_gather_t(
        user_emb.T, item_emb.T, uids.astype(jnp.int32), iids.astype(jnp.int32)
    )
    pred, mse, obj = _fm_call(feat_t, labels, fm_V)
    return pred, obj[0, 0], mse
